# baseline (speedup 1.0000x reference)
import jax
import jax.numpy as jnp
from jax import lax
from jax.experimental import pallas as pl
from jax.experimental.pallas import tpu as pltpu
from jax.experimental.pallas import tpu_sc as plsc

BATCH = 16384
DIMS = 64

_info = plsc.get_sparse_core_info()
NC = _info.num_cores
NS = _info.num_subcores
NW = NC * NS

B_PER_W = BATCH // NW
CHUNK = 128
N_CHUNKS = B_PER_W // CHUNK


def _gmf_body(u_hbm, m_hbm, table_hbm, out_hbm,
              idx_u, idx_m, rows_u, rows_m, sem_u, sem_m):
    wid = lax.axis_index("s") * NC + lax.axis_index("c")
    base = wid * B_PER_W

    for k in range(N_CHUNKS):
        pltpu.sync_copy(u_hbm.at[pl.ds(base + k * CHUNK, CHUNK)], idx_u.at[k])
        pltpu.sync_copy(m_hbm.at[pl.ds(base + k * CHUNK, CHUNK)], idx_m.at[k])

    for k in range(N_CHUNKS):
        cp_u = pltpu.async_copy(table_hbm.at[idx_u.at[k]], rows_u, sem_u)
        cp_m = pltpu.async_copy(table_hbm.at[idx_m.at[k]], rows_m, sem_m)
        cp_u.wait()
        cp_m.wait()

        def mul_row(i, _):
            for j in range(DIMS // 16):
                sl = pl.ds(j * 16, 16)
                rows_u[i, sl] = rows_u[i, sl] * rows_m[i, sl]
            return 0

        lax.fori_loop(0, CHUNK, mul_row, 0)

        pltpu.sync_copy(rows_u, out_hbm.at[pl.ds(base + k * CHUNK, CHUNK)])


@jax.jit
def _gmf(u, m, user_table):
    kfn = pl.kernel(
        _gmf_body,
        out_type=jax.ShapeDtypeStruct((BATCH, DIMS), jnp.float32),
        mesh=plsc.VectorSubcoreMesh(core_axis_name="c", subcore_axis_name="s"),
        compiler_params=pltpu.CompilerParams(use_tc_tiling_on_sc=False),
        scratch_types=[
            pltpu.VMEM((N_CHUNKS, CHUNK), jnp.int32),
            pltpu.VMEM((N_CHUNKS, CHUNK), jnp.int32),
            pltpu.VMEM((CHUNK, DIMS), jnp.float32),
            pltpu.VMEM((CHUNK, DIMS), jnp.float32),
            pltpu.SemaphoreType.DMA,
            pltpu.SemaphoreType.DMA,
        ],
    )
    return kfn(u, m, user_table)


def kernel(u, m, user_table, movie_table):
    return _gmf(u, m, user_table)
